# baseline (speedup 1.0000x reference)
import math

import jax
import jax.numpy as jnp
from jax.experimental import pallas as pl
from jax.experimental.pallas import tpu as pltpu
from jax.experimental.pallas import tpu_sc as plsc

D_MODEL = 64
WINDOW = 128
LANES = 16
SCALE = math.sqrt(D_MODEL)


def _gather_scale(x_flat, W):
    n = x_flat.shape[0]
    num_steps = n // WINDOW
    mesh = plsc.VectorSubcoreMesh(core_axis_name="c", subcore_axis_name="s")
    idx2d = x_flat.reshape(1, n)

    @pl.kernel(
        out_type=jax.ShapeDtypeStruct((n, D_MODEL), jnp.float32),
        mesh=mesh,
        compiler_params=pltpu.CompilerParams(use_tc_tiling_on_sc=False),
    )
    def kern(w_hbm, i_hbm, o_hbm):
        def body(i_vmem, o_vmem):
            pltpu.sync_copy(w_hbm.at[i_vmem.at[0]], o_vmem)

            @pl.loop(0, WINDOW)
            def _(r):
                for c in range(0, D_MODEL, LANES):
                    o_vmem[r, pl.ds(c, LANES)] = (
                        o_vmem[r, pl.ds(c, LANES)] * SCALE
                    )

        pltpu.emit_pipeline(
            body,
            grid=(num_steps,),
            in_specs=[
                pl.BlockSpec((1, WINDOW), index_map=lambda i: (0, i)),
            ],
            out_specs=[
                pl.BlockSpec((WINDOW, D_MODEL), index_map=lambda i: (i, 0)),
            ],
            core_axis_name=("c", "s"),
            dimension_semantics=(pltpu.PARALLEL,),
        )(i_hbm, o_hbm)

    return kern(W, idx2d)


@jax.jit
def kernel(x, W):
    batch, hist = x.shape
    out = _gather_scale(x.reshape(batch * hist), W)
    return out.reshape(batch, hist, D_MODEL)


# speedup vs baseline: 1.4943x; 1.4943x over previous
import math

import jax
import jax.numpy as jnp
from jax import lax
from jax.experimental import pallas as pl
from jax.experimental.pallas import tpu as pltpu
from jax.experimental.pallas import tpu_sc as plsc

D_MODEL = 64
LANES = 16
SCALE = math.sqrt(D_MODEL)

NC, NS = 2, 16
NW = NC * NS
C = 128
NBUF = 8
HALF = NBUF // 2


def _gather_scale(x_flat, W):
    n = x_flat.shape[0]
    rows_w = n // NW
    nchunk = rows_w // C
    nrounds = (nchunk - 2 * HALF) // NBUF
    mesh = plsc.VectorSubcoreMesh(core_axis_name="c", subcore_axis_name="s")

    @pl.kernel(
        out_type=jax.ShapeDtypeStruct((n, D_MODEL), jnp.float32),
        mesh=mesh,
        scratch_types=[
            pltpu.VMEM((rows_w,), jnp.int32),
            *[pltpu.VMEM((C, D_MODEL), jnp.float32) for _ in range(NBUF)],
            pltpu.SemaphoreType.DMA((NBUF,)),
            pltpu.SemaphoreType.DMA((NBUF,)),
            pltpu.SemaphoreType.DMA,
        ],
        compiler_params=pltpu.CompilerParams(use_tc_tiling_on_sc=False),
    )
    def kern(w_hbm, i_hbm, o_hbm, idx_v, *rest):
        bufs = rest[:NBUF]
        gsem, osem, isem = rest[NBUF:]
        wid = lax.axis_index("s") * NC + lax.axis_index("c")
        base = wid * rows_w
        pltpu.async_copy(i_hbm.at[pl.ds(base, rows_w)], idx_v, isem).wait()

        def gdesc(c, b):
            return pltpu.make_async_copy(
                w_hbm.at[idx_v.at[pl.ds(c * C, C)]], bufs[b], gsem.at[b]
            )

        def odesc(c, b):
            return pltpu.make_async_copy(
                bufs[b], o_hbm.at[pl.ds(base + c * C, C)], osem.at[b]
            )

        def scale_buf(b):
            buf = bufs[b]

            @pl.loop(0, C)
            def _(r):
                for col in range(0, D_MODEL, LANES):
                    buf[r, pl.ds(col, LANES)] = (
                        buf[r, pl.ds(col, LANES)] * SCALE
                    )

        for c in range(HALF):
            gdesc(c, c).start()

        for c in range(HALF):
            gdesc(c, c).wait()
            scale_buf(c)
            odesc(c, c).start()
            gdesc(c + HALF, c + HALF).start()

        @pl.loop(0, nrounds)
        def _(r):
            for b in range(NBUF):
                c = HALF + r * NBUF + b
                bb = (HALF + b) % NBUF
                gdesc(c, bb).wait()
                scale_buf(bb)
                odesc(c, bb).start()
                pf = (bb + HALF) % NBUF
                odesc(c - HALF, pf).wait()
                gdesc(c + HALF, pf).start()

        for j in range(HALF):
            c = nchunk - HALF + j
            bb = c % NBUF
            gdesc(c, bb).wait()
            scale_buf(bb)
            odesc(c, bb).start()

        for j in range(NBUF):
            c = nchunk - NBUF + j
            odesc(c, c % NBUF).wait()

    return kern(W, x_flat)


@jax.jit
def kernel(x, W):
    batch, hist = x.shape
    out = _gather_scale(x.reshape(batch * hist), W)
    return out.reshape(batch, hist, D_MODEL)
